# baseline (speedup 1.0000x reference)
import functools

import jax
import jax.numpy as jnp
from jax import lax
from jax.experimental import pallas as pl
from jax.experimental.pallas import tpu as pltpu
from jax.experimental.pallas import tpu_sc as plsc

NC = 2
NS = 16
L = 16
NW = NC * NS

IDX_CHUNK = 128


def _fused_body(b_per_w, n_chunk, d,
                short_hbm, ids_hbm, table_hbm, alpha_hbm, out_hbm,
                idx_v, rows_v, short_v, alpha_v, sem):
    wid = lax.axis_index("s") * NC + lax.axis_index("c")
    base = wid * b_per_w

    for j in range(n_chunk):
        pltpu.sync_copy(ids_hbm.at[pl.ds(base + j * IDX_CHUNK, IDX_CHUNK)],
                        idx_v.at[j])
    copies = [
        pltpu.async_copy(table_hbm.at[idx_v.at[j]],
                         rows_v.at[pl.ds(j * IDX_CHUNK, IDX_CHUNK)], sem)
        for j in range(n_chunk)
    ]
    pltpu.sync_copy(short_hbm.at[pl.ds(base, b_per_w)], short_v)
    pltpu.sync_copy(alpha_hbm, alpha_v)
    for c in copies:
        c.wait()

    a = 1.0 / (1.0 + jnp.exp(-alpha_v[...]))
    om_a = 1.0 - a

    def body(i, carry):
        for dj in range(d // L):
            sl = pl.ds(dj * L, L)
            rows_v[i, sl] = a * short_v[i, sl] + om_a * rows_v[i, sl]
        return carry

    lax.fori_loop(0, b_per_w, body, 0, unroll=4)

    pltpu.sync_copy(rows_v, out_hbm.at[pl.ds(base, b_per_w)])


def kernel(short_term_interest, user_ids, long_term_emb, alpha):
    b, d = short_term_interest.shape
    b_per_w = b // NW
    n_chunk = b_per_w // IDX_CHUNK

    ids = user_ids.astype(jnp.int32)
    alpha_vec = jnp.broadcast_to(jnp.asarray(alpha, jnp.float32).reshape(()), (L,))

    mesh = plsc.VectorSubcoreMesh(core_axis_name="c", subcore_axis_name="s",
                                  num_cores=NC, num_subcores=NS)
    fused = functools.partial(
        pl.kernel,
        out_type=jax.ShapeDtypeStruct((b, d), jnp.float32),
        mesh=mesh,
        scratch_types=[
            pltpu.VMEM((n_chunk, IDX_CHUNK), jnp.int32),
            pltpu.VMEM((b_per_w, d), jnp.float32),
            pltpu.VMEM((b_per_w, d), jnp.float32),
            pltpu.VMEM((L,), jnp.float32),
            pltpu.SemaphoreType.DMA,
        ],
        compiler_params=pltpu.CompilerParams(use_tc_tiling_on_sc=False),
    )(functools.partial(_fused_body, b_per_w, n_chunk, d))
    return fused(short_term_interest, ids, long_term_emb, alpha_vec)


# speedup vs baseline: 2.4978x; 2.4978x over previous
import functools

import jax
import jax.numpy as jnp
from jax import lax
from jax.experimental import pallas as pl
from jax.experimental.pallas import tpu as pltpu
from jax.experimental.pallas import tpu_sc as plsc

NC = 2
NS = 16
L = 16
NW = NC * NS

SUB = 8
SGRP = 64


def _fused_body(b_per_w, d,
                short_hbm, ids_hbm, table_hbm, alpha_hbm, out_hbm,
                ids_v, rows_v, short_a, short_b, alpha_v, sem, ssem):
    wid = lax.axis_index("s") * NC + lax.axis_index("c")
    base = wid * b_per_w

    pltpu.sync_copy(ids_hbm.at[pl.ds(base, b_per_w)], ids_v)
    pltpu.sync_copy(alpha_hbm, alpha_v)

    lanes = lax.iota(jnp.int32, L)

    def issue(g, carry):
        v = ids_v[pl.ds(g * L, L)]
        for l in range(L):
            row = jnp.max(jnp.where(lanes == l, v, 0))
            blk = row // SUB
            rem = lax.rem(row, SUB)
            jb = (g * L + l) // SUB
            pltpu.async_copy(
                table_hbm.at[pl.ds(blk, 1), pl.ds(rem, 1)],
                rows_v.at[pl.ds(jb, 1), pl.ds(l % SUB, 1)], sem)
        return carry

    lax.fori_loop(0, b_per_w // L, issue, 0)

    sbufs = [short_a, short_b]
    n_grp = b_per_w // SGRP
    c0 = pltpu.async_copy(
        short_hbm.at[pl.ds(base // SUB, SGRP // SUB)], sbufs[0], ssem)
    pltpu.make_async_copy(table_hbm.at[pl.ds(0, b_per_w // SUB)],
                          rows_v, sem).wait()

    a = 1.0 / (1.0 + jnp.exp(-alpha_v[...]))
    om_a = 1.0 - a

    pending = c0
    for grp in range(n_grp):
        pending.wait()
        if grp + 1 < n_grp:
            pending = pltpu.async_copy(
                short_hbm.at[pl.ds((base + (grp + 1) * SGRP) // SUB,
                                   SGRP // SUB)],
                sbufs[(grp + 1) % 2], ssem)
        sbuf = sbufs[grp % 2]

        def body(j, carry, grp=grp, sbuf=sbuf):
            jb = grp * (SGRP // SUB) + j // SUB
            sb = j // SUB
            js = lax.rem(j, SUB)
            for dj in range(d // L):
                sl = pl.ds(dj * L, L)
                r = rows_v[jb, js, sl]
                s = sbuf[sb, js, sl]
                rows_v[jb, js, sl] = a * s + om_a * r
            return carry

        lax.fori_loop(0, SGRP, body, 0, unroll=2)

    pltpu.sync_copy(rows_v, out_hbm.at[pl.ds(base // SUB, b_per_w // SUB)])


def kernel(short_term_interest, user_ids, long_term_emb, alpha):
    b, d = short_term_interest.shape
    b_per_w = b // NW

    ids = user_ids.astype(jnp.int32)
    alpha_vec = jnp.broadcast_to(jnp.asarray(alpha, jnp.float32).reshape(()), (L,))
    short_t = short_term_interest.reshape(b // SUB, SUB, d)
    table_t = long_term_emb.reshape(long_term_emb.shape[0] // SUB, SUB, d)

    mesh = plsc.VectorSubcoreMesh(core_axis_name="c", subcore_axis_name="s",
                                  num_cores=NC, num_subcores=NS)
    fused = functools.partial(
        pl.kernel,
        out_type=jax.ShapeDtypeStruct((b // SUB, SUB, d), jnp.float32),
        mesh=mesh,
        scratch_types=[
            pltpu.VMEM((b_per_w,), jnp.int32),
            pltpu.VMEM((b_per_w // SUB, SUB, d), jnp.float32),
            pltpu.VMEM((SGRP // SUB, SUB, d), jnp.float32),
            pltpu.VMEM((SGRP // SUB, SUB, d), jnp.float32),
            pltpu.VMEM((L,), jnp.float32),
            pltpu.SemaphoreType.DMA,
            pltpu.SemaphoreType.DMA,
        ],
        compiler_params=pltpu.CompilerParams(needs_layout_passes=False),
    )(functools.partial(_fused_body, b_per_w, d))
    out_t = fused(short_t, ids, table_t, alpha_vec)
    return out_t.reshape(b, d)
